# Initial kernel scaffold; baseline (speedup 1.0000x reference)
import jax
import jax.numpy as jnp
from jax.experimental import pallas as pl


def kernel(tgt, features):
    raise NotImplementedError("write your pallas kernel here")


# speedup vs baseline: 1.7664x; 1.7664x over previous
import functools

import jax
import jax.numpy as jnp
import numpy as np
from jax import lax
from jax.experimental import pallas as pl
from jax.experimental.pallas import tpu as pltpu
from jax.experimental.pallas import tpu_sc as plsc

_N = 1440
_FDIM = 16
_B = 1048576
_ROWS = _B // 128
_TC_BLK = 1024
_NC, _NS = 2, 16
_NW = _NC * _NS
_BW = _B // _NW
_CH = 512
_G = _CH // 128
_CHUNKS = _BW // _CH
_ROWS_W = _BW // 128

_STEP = np.float32(np.deg2rad(0.25))
_TWO_PI = np.float32(2.0 * np.pi)
_EPS = np.float32(1e-8)


def _prep_body(x_ref, y_ref, z_ref, idx_ref, w_ref):
    x = x_ref[...]
    y = y_ref[...]
    z = z_ref[...]
    a = jnp.arctan2(y, x)
    theta = jnp.where(a < 0, a + _TWO_PI, a)
    r = jnp.sqrt(x * x + y * y) + _EPS
    b = jnp.arctan2(z, r)
    phi = jnp.where(b < 0, b + _TWO_PI, b)
    u = theta / _STEP
    v = phi / _STEP
    i0f = jnp.floor(u)
    j0f = jnp.floor(v)
    du = u - i0f
    dv = v - j0f
    i0 = i0f.astype(jnp.int32)
    j0 = j0f.astype(jnp.int32)
    i0 = jnp.where(i0 >= _N, i0 - _N, i0)
    j0 = jnp.where(j0 >= _N, j0 - _N, j0)
    i1 = jnp.where(i0 + 1 == _N, 0, i0 + 1)
    j1 = jnp.where(j0 + 1 == _N, 0, j0 + 1)
    idx_ref[0] = i0 * _N + j0
    idx_ref[1] = i1 * _N + j0
    idx_ref[2] = i0 * _N + j1
    idx_ref[3] = i1 * _N + j1
    w_ref[0] = (1.0 - du) * (1.0 - dv)
    w_ref[1] = du * (1.0 - dv)
    w_ref[2] = (1.0 - du) * dv
    w_ref[3] = du * dv


def _prep(x2, y2, z2):
    in_spec = pl.BlockSpec((_TC_BLK, 128), lambda i: (i, 0))
    out_spec = pl.BlockSpec((4, _TC_BLK, 128), lambda i: (0, i, 0))
    return pl.pallas_call(
        _prep_body,
        grid=(_ROWS // _TC_BLK,),
        in_specs=[in_spec] * 3,
        out_specs=[out_spec, out_spec],
        out_shape=[
            jax.ShapeDtypeStruct((4, _ROWS, 128), jnp.int32),
            jax.ShapeDtypeStruct((4, _ROWS, 128), jnp.float32),
        ],
    )(x2, y2, z2)


def _sc_body(idx_h, w_h, ftab_h, out_h, idx_v, w_v, r00_v, r10_v, r01_v,
             r11_v, out_v, sem):
    wid = lax.axis_index("s") * _NC + lax.axis_index("c")
    row0 = wid * _ROWS_W
    iota16 = lax.iota(jnp.int32, 16)

    def chunk_body(k, carry):
        rbase = row0 + k * _G
        pltpu.sync_copy(idx_h.at[:, pl.ds(rbase, _G), :], idx_v)
        pltpu.sync_copy(w_h.at[:, pl.ds(rbase, _G), :], w_v)
        descs = []
        for g in range(_G):
            descs.append(pltpu.async_copy(
                ftab_h.at[idx_v.at[0, g]], r00_v.at[pl.ds(g * 128, 128)], sem))
            descs.append(pltpu.async_copy(
                ftab_h.at[idx_v.at[1, g]], r10_v.at[pl.ds(g * 128, 128)], sem))
            descs.append(pltpu.async_copy(
                ftab_h.at[idx_v.at[2, g]], r01_v.at[pl.ds(g * 128, 128)], sem))
            descs.append(pltpu.async_copy(
                ftab_h.at[idx_v.at[3, g]], r11_v.at[pl.ds(g * 128, 128)], sem))
        for d in descs:
            d.wait()

        def gg_body(gg, c2):
            for t in range(128 // 16):
                q0 = gg * 128 + t * 16
                w00 = w_v[0, gg, pl.ds(t * 16, 16)]
                w10 = w_v[1, gg, pl.ds(t * 16, 16)]
                w01 = w_v[2, gg, pl.ds(t * 16, 16)]
                w11 = w_v[3, gg, pl.ds(t * 16, 16)]
                for l in range(16):
                    q = q0 + l
                    a00 = jnp.broadcast_to(w00[l], (16,))
                    a10 = jnp.broadcast_to(w10[l], (16,))
                    a01 = jnp.broadcast_to(w01[l], (16,))
                    a11 = jnp.broadcast_to(w11[l], (16,))
                    f00 = r00_v[q, :]
                    f10 = r10_v[q, :]
                    f01 = r01_v[q, :]
                    f11 = r11_v[q, :]
                    out_v[q, :] = f00 * a00 + f10 * a10 + f01 * a01 + f11 * a11
            return c2

        lax.fori_loop(0, _G, gg_body, 0)
        qbase = wid * _BW + k * _CH
        pltpu.sync_copy(out_v, out_h.at[pl.ds(qbase, _CH)])
        return carry

    lax.fori_loop(0, _CHUNKS, chunk_body, 0)


@functools.lru_cache(maxsize=1)
def _get_sc_interp():
    return functools.partial(
        pl.kernel,
        mesh=plsc.VectorSubcoreMesh(core_axis_name="c", subcore_axis_name="s"),
        out_type=jax.ShapeDtypeStruct((_B, _FDIM), jnp.float32),
        compiler_params=pltpu.CompilerParams(use_tc_tiling_on_sc=False),
        scratch_types=[
            pltpu.VMEM((4, _G, 128), jnp.int32),
            pltpu.VMEM((4, _G, 128), jnp.float32),
            pltpu.VMEM((_CH, _FDIM), jnp.float32),
            pltpu.VMEM((_CH, _FDIM), jnp.float32),
            pltpu.VMEM((_CH, _FDIM), jnp.float32),
            pltpu.VMEM((_CH, _FDIM), jnp.float32),
            pltpu.VMEM((_CH, _FDIM), jnp.float32),
            pltpu.SemaphoreType.DMA,
        ],
    )(_sc_body)


def kernel(tgt, features):
    x2 = tgt[:, 0].reshape(_ROWS, 128)
    y2 = tgt[:, 1].reshape(_ROWS, 128)
    z2 = tgt[:, 2].reshape(_ROWS, 128)
    idx_all, w_all = _prep(x2, y2, z2)
    ftab = features.reshape(_N * _N, _FDIM)
    return _get_sc_interp()(idx_all, w_all, ftab)


# speedup vs baseline: 1.7672x; 1.0005x over previous
import functools

import jax
import jax.numpy as jnp
import numpy as np
from jax import lax
from jax.experimental import pallas as pl
from jax.experimental.pallas import tpu as pltpu
from jax.experimental.pallas import tpu_sc as plsc

_N = 1440
_FDIM = 16
_B = 1048576
_ROWS = _B // 128
_TC_BLK = 1024
_NC, _NS = 2, 16
_NW = _NC * _NS
_BW = _B // _NW
_CH = 512
_G = _CH // 128
_CHUNKS = _BW // _CH
_ROWS_W = _BW // 128

_STEP = np.float32(np.deg2rad(0.25))
_TWO_PI = np.float32(2.0 * np.pi)
_EPS = np.float32(1e-8)


def _prep_body(x_ref, y_ref, z_ref, idx_ref, w_ref):
    x = x_ref[...]
    y = y_ref[...]
    z = z_ref[...]
    a = jnp.arctan2(y, x)
    theta = jnp.where(a < 0, a + _TWO_PI, a)
    r = jnp.sqrt(x * x + y * y) + _EPS
    b = jnp.arctan2(z, r)
    phi = jnp.where(b < 0, b + _TWO_PI, b)
    u = theta / _STEP
    v = phi / _STEP
    i0f = jnp.floor(u)
    j0f = jnp.floor(v)
    du = u - i0f
    dv = v - j0f
    i0 = i0f.astype(jnp.int32)
    j0 = j0f.astype(jnp.int32)
    i0 = jnp.where(i0 >= _N, i0 - _N, i0)
    j0 = jnp.where(j0 >= _N, j0 - _N, j0)
    i1 = jnp.where(i0 + 1 == _N, 0, i0 + 1)
    j1 = jnp.where(j0 + 1 == _N, 0, j0 + 1)
    idx_ref[0] = i0 * _N + j0
    idx_ref[1] = i1 * _N + j0
    idx_ref[2] = i0 * _N + j1
    idx_ref[3] = i1 * _N + j1
    w_ref[0] = (1.0 - du) * (1.0 - dv)
    w_ref[1] = du * (1.0 - dv)
    w_ref[2] = (1.0 - du) * dv
    w_ref[3] = du * dv


def _prep(x2, y2, z2):
    in_spec = pl.BlockSpec((_TC_BLK, 128), lambda i: (i, 0))
    out_spec = pl.BlockSpec((4, _TC_BLK, 128), lambda i: (0, i, 0))
    return pl.pallas_call(
        _prep_body,
        grid=(_ROWS // _TC_BLK,),
        in_specs=[in_spec] * 3,
        out_specs=[out_spec, out_spec],
        out_shape=[
            jax.ShapeDtypeStruct((4, _ROWS, 128), jnp.int32),
            jax.ShapeDtypeStruct((4, _ROWS, 128), jnp.float32),
        ],
    )(x2, y2, z2)


def _sc_body(idx_h, w_h, ftab_h, out_h, idx_v, w_v, r00_v, r10_v, r01_v,
             r11_v, out_v, sem):
    wid = lax.axis_index("s") * _NC + lax.axis_index("c")
    row0 = wid * _ROWS_W

    def chunk_body(k, carry):
        rbase = row0 + k * _G
        pltpu.sync_copy(idx_h.at[:, pl.ds(rbase, _G), :], idx_v)
        pltpu.sync_copy(w_h.at[:, pl.ds(rbase, _G), :], w_v)
        descs = []
        for g in range(_G):
            descs.append(pltpu.async_copy(
                ftab_h.at[idx_v.at[0, g]], r00_v.at[pl.ds(g * 128, 128)], sem))
            descs.append(pltpu.async_copy(
                ftab_h.at[idx_v.at[1, g]], r10_v.at[pl.ds(g * 128, 128)], sem))
            descs.append(pltpu.async_copy(
                ftab_h.at[idx_v.at[2, g]], r01_v.at[pl.ds(g * 128, 128)], sem))
            descs.append(pltpu.async_copy(
                ftab_h.at[idx_v.at[3, g]], r11_v.at[pl.ds(g * 128, 128)], sem))
        for d in descs:
            d.wait()

        def gg_body(gg, c2):
            for t in range(128 // 16):
                w00 = w_v[0, gg, pl.ds(t * 16, 16)]
                w10 = w_v[1, gg, pl.ds(t * 16, 16)]
                w01 = w_v[2, gg, pl.ds(t * 16, 16)]
                w11 = w_v[3, gg, pl.ds(t * 16, 16)]
                for l in range(16):
                    q = gg * 128 + t * 16 + l
                    a00 = jnp.broadcast_to(w00[l], (16,))
                    a10 = jnp.broadcast_to(w10[l], (16,))
                    a01 = jnp.broadcast_to(w01[l], (16,))
                    a11 = jnp.broadcast_to(w11[l], (16,))
                    out_v[q, :] = (r00_v[q, :] * a00 + r10_v[q, :] * a10
                                   + r01_v[q, :] * a01 + r11_v[q, :] * a11)
            return c2

        lax.fori_loop(0, _G, gg_body, 0)
        pltpu.sync_copy(out_v, out_h.at[pl.ds(rbase * 128, _CH)])
        return carry

    lax.fori_loop(0, _CHUNKS, chunk_body, 0)


@functools.lru_cache(maxsize=1)
def _get_sc_interp():
    return functools.partial(
        pl.kernel,
        mesh=plsc.VectorSubcoreMesh(core_axis_name="c", subcore_axis_name="s"),
        out_type=jax.ShapeDtypeStruct((_B, _FDIM), jnp.float32),
        compiler_params=pltpu.CompilerParams(use_tc_tiling_on_sc=False),
        scratch_types=[
            pltpu.VMEM((4, _G, 128), jnp.int32),
            pltpu.VMEM((4, _G, 128), jnp.float32),
            pltpu.VMEM((_CH, _FDIM), jnp.float32),
            pltpu.VMEM((_CH, _FDIM), jnp.float32),
            pltpu.VMEM((_CH, _FDIM), jnp.float32),
            pltpu.VMEM((_CH, _FDIM), jnp.float32),
            pltpu.VMEM((_CH, _FDIM), jnp.float32),
            pltpu.SemaphoreType.DMA,
        ],
    )(_sc_body)


def kernel(tgt, features):
    tgt_t = jnp.transpose(tgt)
    x2 = tgt_t[0].reshape(_ROWS, 128)
    y2 = tgt_t[1].reshape(_ROWS, 128)
    z2 = tgt_t[2].reshape(_ROWS, 128)
    idx_all, w_all = _prep(x2, y2, z2)
    ftab = features.reshape(_N * _N, _FDIM)
    return _get_sc_interp()(idx_all, w_all, ftab)


# speedup vs baseline: 1.9623x; 1.1104x over previous
import functools

import jax
import jax.numpy as jnp
import numpy as np
from jax import lax
from jax.experimental import pallas as pl
from jax.experimental.pallas import tpu as pltpu
from jax.experimental.pallas import tpu_sc as plsc

_N = 1440
_FDIM = 16
_B = 1048576
_ROWS = _B // 128
_TC_BLK = 1024
_NC, _NS = 2, 16
_NW = _NC * _NS
_BW = _B // _NW
_CH = 512
_G = _CH // 128
_CHUNKS = _BW // _CH
_ROWS_W = _BW // 128

_STEP = np.float32(np.deg2rad(0.25))
_TWO_PI = np.float32(2.0 * np.pi)
_EPS = np.float32(1e-8)


def _prep_body(x_ref, y_ref, z_ref, idx_ref, w_ref):
    x = x_ref[...]
    y = y_ref[...]
    z = z_ref[...]
    a = jnp.arctan2(y, x)
    theta = jnp.where(a < 0, a + _TWO_PI, a)
    r = jnp.sqrt(x * x + y * y) + _EPS
    b = jnp.arctan2(z, r)
    phi = jnp.where(b < 0, b + _TWO_PI, b)
    u = theta / _STEP
    v = phi / _STEP
    i0f = jnp.floor(u)
    j0f = jnp.floor(v)
    du = u - i0f
    dv = v - j0f
    i0 = i0f.astype(jnp.int32)
    j0 = j0f.astype(jnp.int32)
    i0 = jnp.where(i0 >= _N, i0 - _N, i0)
    j0 = jnp.where(j0 >= _N, j0 - _N, j0)
    i1 = jnp.where(i0 + 1 == _N, 0, i0 + 1)
    j1 = jnp.where(j0 + 1 == _N, 0, j0 + 1)
    idx_ref[0] = i0 * _N + j0
    idx_ref[1] = i1 * _N + j0
    idx_ref[2] = i0 * _N + j1
    idx_ref[3] = i1 * _N + j1
    w_ref[0] = (1.0 - du) * (1.0 - dv)
    w_ref[1] = du * (1.0 - dv)
    w_ref[2] = (1.0 - du) * dv
    w_ref[3] = du * dv


def _prep(x2, y2, z2):
    in_spec = pl.BlockSpec((_TC_BLK, 128), lambda i: (i, 0))
    out_spec = pl.BlockSpec((4, _TC_BLK, 128), lambda i: (0, i, 0))
    return pl.pallas_call(
        _prep_body,
        grid=(_ROWS // _TC_BLK,),
        in_specs=[in_spec] * 3,
        out_specs=[out_spec, out_spec],
        out_shape=[
            jax.ShapeDtypeStruct((4, _ROWS, 128), jnp.int32),
            jax.ShapeDtypeStruct((4, _ROWS, 128), jnp.float32),
        ],
    )(x2, y2, z2)


def _sc_body(idx_h, w_h, ftab_h, out_h, idx_v, w_v, r00_v, r10_v, r01_v,
             r11_v, out_v, gsem0, gsem1):
    wid = lax.axis_index("s") * _NC + lax.axis_index("c")
    row0 = wid * _ROWS_W
    gsems = (gsem0, gsem1)

    def stage(b, rbase):
        pltpu.sync_copy(idx_h.at[:, pl.ds(rbase, _G), :], idx_v.at[b])
        pltpu.sync_copy(w_h.at[:, pl.ds(rbase, _G), :], w_v.at[b])
        for g in range(_G):
            sl = pl.ds(g * 128, 128)
            pltpu.async_copy(ftab_h.at[idx_v.at[b, 0, g]], r00_v.at[b, sl],
                             gsems[b])
            pltpu.async_copy(ftab_h.at[idx_v.at[b, 1, g]], r10_v.at[b, sl],
                             gsems[b])
            pltpu.async_copy(ftab_h.at[idx_v.at[b, 2, g]], r01_v.at[b, sl],
                             gsems[b])
            pltpu.async_copy(ftab_h.at[idx_v.at[b, 3, g]], r11_v.at[b, sl],
                             gsems[b])

    def drain(b):
        for rv in (r00_v, r10_v, r01_v, r11_v):
            pltpu.make_async_copy(ftab_h.at[pl.ds(0, _CH)], rv.at[b],
                                  gsems[b]).wait()

    def compute(b, rbase):
        def gg_body(gg, c2):
            for t in range(128 // 16):
                w00 = w_v[b, 0, gg, pl.ds(t * 16, 16)]
                w10 = w_v[b, 1, gg, pl.ds(t * 16, 16)]
                w01 = w_v[b, 2, gg, pl.ds(t * 16, 16)]
                w11 = w_v[b, 3, gg, pl.ds(t * 16, 16)]
                for l in range(16):
                    q = gg * 128 + t * 16 + l
                    a00 = jnp.broadcast_to(w00[l], (16,))
                    a10 = jnp.broadcast_to(w10[l], (16,))
                    a01 = jnp.broadcast_to(w01[l], (16,))
                    a11 = jnp.broadcast_to(w11[l], (16,))
                    out_v[q, :] = (r00_v[b, q, :] * a00 + r10_v[b, q, :] * a10
                                   + r01_v[b, q, :] * a01
                                   + r11_v[b, q, :] * a11)
            return c2

        lax.fori_loop(0, _G, gg_body, 0)
        pltpu.sync_copy(out_v, out_h.at[pl.ds(rbase * 128, _CH)])

    stage(0, row0)

    def pair_body(kk, carry):
        rb = row0 + (2 * kk) * _G
        stage(1, rb + _G)
        drain(0)
        compute(0, rb)
        stage(0, rb + 2 * _G)
        drain(1)
        compute(1, rb + _G)
        return carry

    lax.fori_loop(0, _CHUNKS // 2 - 1, pair_body, 0)
    rb_last = row0 + (_CHUNKS - 2) * _G
    stage(1, rb_last + _G)
    drain(0)
    compute(0, rb_last)
    drain(1)
    compute(1, rb_last + _G)


@functools.lru_cache(maxsize=1)
def _get_sc_interp():
    return functools.partial(
        pl.kernel,
        mesh=plsc.VectorSubcoreMesh(core_axis_name="c", subcore_axis_name="s"),
        out_type=jax.ShapeDtypeStruct((_B, _FDIM), jnp.float32),
        compiler_params=pltpu.CompilerParams(use_tc_tiling_on_sc=False),
        scratch_types=[
            pltpu.VMEM((2, 4, _G, 128), jnp.int32),
            pltpu.VMEM((2, 4, _G, 128), jnp.float32),
            pltpu.VMEM((2, _CH, _FDIM), jnp.float32),
            pltpu.VMEM((2, _CH, _FDIM), jnp.float32),
            pltpu.VMEM((2, _CH, _FDIM), jnp.float32),
            pltpu.VMEM((2, _CH, _FDIM), jnp.float32),
            pltpu.VMEM((_CH, _FDIM), jnp.float32),
            pltpu.SemaphoreType.DMA,
            pltpu.SemaphoreType.DMA,
        ],
    )(_sc_body)


def kernel(tgt, features):
    tgt_t = jnp.transpose(tgt)
    x2 = tgt_t[0].reshape(_ROWS, 128)
    y2 = tgt_t[1].reshape(_ROWS, 128)
    z2 = tgt_t[2].reshape(_ROWS, 128)
    idx_all, w_all = _prep(x2, y2, z2)
    ftab = features.reshape(_N * _N, _FDIM)
    return _get_sc_interp()(idx_all, w_all, ftab)


# speedup vs baseline: 2.6501x; 1.3505x over previous
import functools

import jax
import jax.numpy as jnp
import numpy as np
from jax import lax
from jax.experimental import pallas as pl
from jax.experimental.pallas import tpu as pltpu
from jax.experimental.pallas import tpu_sc as plsc

_N = 1440
_FDIM = 16
_B = 1048576
_ROWS = _B // 128
_TC_BLK = 1024
_NC, _NS = 2, 16
_NW = _NC * _NS
_BW = _B // _NW
_CH = 512
_G = _CH // 128
_CHUNKS = _BW // _CH
_ROWS_W = _BW // 128

_STEP = np.float32(np.deg2rad(0.25))
_TWO_PI = np.float32(2.0 * np.pi)
_EPS = np.float32(1e-8)


def _prep_body(x_ref, y_ref, z_ref, idx_ref, w_ref):
    x = x_ref[...]
    y = y_ref[...]
    z = z_ref[...]
    a = jnp.arctan2(y, x)
    theta = jnp.where(a < 0, a + _TWO_PI, a)
    r = jnp.sqrt(x * x + y * y) + _EPS
    b = jnp.arctan2(z, r)
    phi = jnp.where(b < 0, b + _TWO_PI, b)
    u = theta / _STEP
    v = phi / _STEP
    i0f = jnp.floor(u)
    j0f = jnp.floor(v)
    du = u - i0f
    dv = v - j0f
    i0 = i0f.astype(jnp.int32)
    j0 = j0f.astype(jnp.int32)
    i0 = jnp.where(i0 >= _N, i0 - _N, i0)
    j0 = jnp.where(j0 >= _N, j0 - _N, j0)
    i1 = jnp.where(i0 + 1 == _N, 0, i0 + 1)
    j1 = jnp.where(j0 + 1 == _N, 0, j0 + 1)
    idx_ref[0] = i0 * _N + j0
    idx_ref[1] = i1 * _N + j0
    idx_ref[2] = i0 * _N + j1
    idx_ref[3] = i1 * _N + j1
    w_ref[0] = (1.0 - du) * (1.0 - dv)
    w_ref[1] = du * (1.0 - dv)
    w_ref[2] = (1.0 - du) * dv
    w_ref[3] = du * dv


def _prep(x2, y2, z2):
    in_spec = pl.BlockSpec((_TC_BLK, 128), lambda i: (i, 0))
    out_spec = pl.BlockSpec((4, _TC_BLK, 128), lambda i: (0, i, 0))
    return pl.pallas_call(
        _prep_body,
        grid=(_ROWS // _TC_BLK,),
        in_specs=[in_spec] * 3,
        out_specs=[out_spec, out_spec],
        out_shape=[
            jax.ShapeDtypeStruct((4, _ROWS, 128), jnp.int32),
            jax.ShapeDtypeStruct((4, _ROWS, 128), jnp.float32),
        ],
    )(x2, y2, z2)


_GATHER_DNUMS = lax.GatherDimensionNumbers(
    offset_dims=(), collapsed_slice_dims=(0,), start_index_map=(0,))


def _lane_perm(v, idx):
    return lax.gather(v, idx[:, None], _GATHER_DNUMS, (1,),
                      mode=lax.GatherScatterMode.PROMISE_IN_BOUNDS)


def _transpose16(vs, lane):
    vs = list(vs)
    for s in (1, 2, 4, 8):
        idx = lane ^ s
        m = (lane & s) != 0
        for r in range(16):
            if r & s:
                continue
            a, b = vs[r], vs[r + s]
            ga = _lane_perm(a, idx)
            gb = _lane_perm(b, idx)
            vs[r] = jnp.where(m, gb, a)
            vs[r + s] = jnp.where(m, b, ga)
    return vs


def _sc_body(idx_h, w_h, ftab_h, out_h, idx_v, w_v, r00_v, r10_v, r01_v,
             r11_v, out_v, gsem0, gsem1, osem):
    wid = lax.axis_index("s") * _NC + lax.axis_index("c")
    row0 = wid * _ROWS_W
    gsems = (gsem0, gsem1)
    lane = lax.iota(jnp.int32, 16)

    def stage(b, rbase):
        pltpu.sync_copy(idx_h.at[:, pl.ds(rbase, _G), :], idx_v.at[b])
        pltpu.sync_copy(w_h.at[:, pl.ds(rbase, _G), :], w_v.at[b])
        for g in range(_G):
            sl = pl.ds(g * 128, 128)
            pltpu.async_copy(ftab_h.at[idx_v.at[b, 0, g]], r00_v.at[b, sl],
                             gsems[b])
            pltpu.async_copy(ftab_h.at[idx_v.at[b, 1, g]], r10_v.at[b, sl],
                             gsems[b])
            pltpu.async_copy(ftab_h.at[idx_v.at[b, 2, g]], r01_v.at[b, sl],
                             gsems[b])
            pltpu.async_copy(ftab_h.at[idx_v.at[b, 3, g]], r11_v.at[b, sl],
                             gsems[b])

    def drain(b):
        for rv in (r00_v, r10_v, r01_v, r11_v):
            pltpu.make_async_copy(ftab_h.at[pl.ds(0, _CH)], rv.at[b],
                                  gsems[b]).wait()

    def compute(b, rbase):
        def gg_body(gg, c2):
            for t in range(128 // 16):
                w00 = w_v[b, 0, gg, pl.ds(t * 16, 16)]
                w10 = w_v[b, 1, gg, pl.ds(t * 16, 16)]
                w01 = w_v[b, 2, gg, pl.ds(t * 16, 16)]
                w11 = w_v[b, 3, gg, pl.ds(t * 16, 16)]
                accs = []
                for l in range(16):
                    q = gg * 128 + t * 16 + l
                    a00 = jnp.broadcast_to(w00[l], (16,))
                    a10 = jnp.broadcast_to(w10[l], (16,))
                    a01 = jnp.broadcast_to(w01[l], (16,))
                    a11 = jnp.broadcast_to(w11[l], (16,))
                    accs.append(r00_v[b, q, :] * a00 + r10_v[b, q, :] * a10
                                + r01_v[b, q, :] * a01 + r11_v[b, q, :] * a11)
                cols = _transpose16(accs, lane)
                for d in range(_FDIM):
                    out_v[d, gg, pl.ds(t * 16, 16)] = cols[d]
            return c2

        lax.fori_loop(0, _G, gg_body, 0)
        odescs = [
            pltpu.async_copy(out_v.at[d],
                             out_h.at[d // 8, pl.ds(rbase, _G), d % 8, :],
                             osem)
            for d in range(_FDIM)
        ]
        for d2 in odescs:
            d2.wait()

    stage(0, row0)

    def pair_body(kk, carry):
        rb = row0 + (2 * kk) * _G
        stage(1, rb + _G)
        drain(0)
        compute(0, rb)
        stage(0, rb + 2 * _G)
        drain(1)
        compute(1, rb + _G)
        return carry

    lax.fori_loop(0, _CHUNKS // 2 - 1, pair_body, 0)
    rb_last = row0 + (_CHUNKS - 2) * _G
    stage(1, rb_last + _G)
    drain(0)
    compute(0, rb_last)
    drain(1)
    compute(1, rb_last + _G)


@functools.lru_cache(maxsize=1)
def _get_sc_interp():
    return functools.partial(
        pl.kernel,
        mesh=plsc.VectorSubcoreMesh(core_axis_name="c", subcore_axis_name="s"),
        out_type=jax.ShapeDtypeStruct((2, _B // 128, 8, 128), jnp.float32),
        compiler_params=pltpu.CompilerParams(use_tc_tiling_on_sc=False),
        scratch_types=[
            pltpu.VMEM((2, 4, _G, 128), jnp.int32),
            pltpu.VMEM((2, 4, _G, 128), jnp.float32),
            pltpu.VMEM((2, _CH, _FDIM), jnp.float32),
            pltpu.VMEM((2, _CH, _FDIM), jnp.float32),
            pltpu.VMEM((2, _CH, _FDIM), jnp.float32),
            pltpu.VMEM((2, _CH, _FDIM), jnp.float32),
            pltpu.VMEM((_FDIM, _G, 128), jnp.float32),
            pltpu.SemaphoreType.DMA,
            pltpu.SemaphoreType.DMA,
            pltpu.SemaphoreType.DMA,
        ],
    )(_sc_body)


def kernel(tgt, features):
    tgt_t = jnp.transpose(tgt)
    x2 = tgt_t[0].reshape(_ROWS, 128)
    y2 = tgt_t[1].reshape(_ROWS, 128)
    z2 = tgt_t[2].reshape(_ROWS, 128)
    idx_all, w_all = _prep(x2, y2, z2)
    ftab = features.reshape(_N * _N, _FDIM)
    out4 = _get_sc_interp()(idx_all, w_all, ftab)
    return jnp.transpose(out4, (1, 3, 0, 2)).reshape(_B, _FDIM)
